# baseline (speedup 1.0000x reference)
import jax
import jax.numpy as jnp
from jax import lax
from jax.experimental import pallas as pl
from jax.experimental.pallas import tpu as pltpu
from jax.experimental.pallas import tpu_sc as plsc

B = 16384
F = 26
DIM = 32
N = B * F
R = 2600001
RPAD = 2600064
NBLK = RPAD // 128
NUM_WORKERS = 32
LANES = 16

PER_W = N // NUM_WORKERS
GSZ = 128
K = 13
STEP = K * GSZ
NSTEPS = PER_W // STEP

TCH = 384
TGRID = RPAD // TCH


def _transpose_body(in_ref, out_ref):
    out_ref[...] = in_ref[...].T


def _emb_body(x_hbm, offrep_hbm, table_hbm, out_hbm,
              off_v, raw_v, idx_v, rows_v, sem):
    cid = lax.axis_index("c")
    sid = lax.axis_index("s")
    wid = sid * 2 + cid
    base = wid * PER_W

    pltpu.sync_copy(offrep_hbm, off_v)

    def step(s, carry):
        s_base = base + s * STEP
        pltpu.sync_copy(x_hbm.at[pl.ds(s_base, STEP)], raw_v)
        for j in range(K):
            for c in range(GSZ // LANES):
                p0 = j * GSZ + c * LANES
                idx_v[j, pl.ds(c * LANES, LANES)] = (
                    raw_v[pl.ds(p0, LANES)] + off_v[pl.ds(p0, LANES)]
                )
        cps = [
            pltpu.async_copy(
                table_hbm.at[idx_v.at[j]],
                rows_v.at[pl.ds(j * GSZ, GSZ)],
                sem,
            )
            for j in range(K)
        ]
        for cp in cps:
            cp.wait()
        pltpu.sync_copy(rows_v, out_hbm.at[pl.ds(s_base, STEP)])
        return carry

    lax.fori_loop(0, NSTEPS, step, 0)


@jax.jit
def _emb_call(x_flat, off_rep, emb_t):
    mesh = plsc.VectorSubcoreMesh(core_axis_name="c", subcore_axis_name="s")
    table = pl.pallas_call(
        _transpose_body,
        grid=(TGRID,),
        in_specs=[pl.BlockSpec((DIM, TCH), lambda i: (0, i))],
        out_specs=pl.BlockSpec((TCH, DIM), lambda i: (i, 0)),
        out_shape=jax.ShapeDtypeStruct((RPAD, DIM), jnp.float32),
    )(emb_t)
    gather = pl.kernel(
        _emb_body,
        out_type=jax.ShapeDtypeStruct((N, DIM), jnp.float32),
        mesh=mesh,
        scratch_types=[
            pltpu.VMEM((STEP,), jnp.int32),
            pltpu.VMEM((STEP,), jnp.int32),
            pltpu.VMEM((K, GSZ), jnp.int32),
            pltpu.VMEM((STEP, DIM), jnp.float32),
            pltpu.SemaphoreType.DMA,
        ],
        compiler_params=pltpu.CompilerParams(use_tc_tiling_on_sc=False),
    )
    return gather(x_flat, off_rep, table)


def kernel(x_cat, emb_weight, offsets):
    x_flat = x_cat.astype(jnp.int32).reshape(N)
    off_rep = jnp.tile(offsets.astype(jnp.int32), STEP // F)
    out = _emb_call(x_flat, off_rep, emb_weight.T)
    return out.reshape(B, F, DIM)


# speedup vs baseline: 1.6027x; 1.6027x over previous
import jax
import jax.numpy as jnp
from jax import lax
from jax.experimental import pallas as pl
from jax.experimental.pallas import tpu as pltpu
from jax.experimental.pallas import tpu_sc as plsc

B = 16384
F = 26
DIM = 32
N = B * F
R = 2600001
RPAD = 2600064
NBLK = RPAD // 128
Q = RPAD // 4
NUM_WORKERS = 32
LANES = 16

NFULL = NBLK - 1
TAIL = R - NFULL * 128
BLK_PER_W = -(-NFULL // NUM_WORKERS)

PER_W = N // NUM_WORKERS
GROUP = 128 * F
NGROUPS = PER_W // GROUP
CHUNK = 832
NCHUNK = GROUP // CHUNK
GSZ = 64
NG = CHUNK // GSZ

_CPARAMS = pltpu.CompilerParams(
    use_tc_tiling_on_sc=True, needs_layout_passes=False)


def _transpose_block(v_in, v_out):
    for r in range(32):
        for k in range(8):
            rows = (k % 2) * LANES + lax.iota(jnp.int32, LANES)
            src = plsc.load_gather(
                v_in, [rows, jnp.full((LANES,), 4 * r + k // 2, jnp.int32)]
            )
            v_out[r, pl.ds(k * LANES, LANES)] = src


def _detile_body(embt_hbm, tail_hbm, out_hbm, v_in, v_out):
    cid = lax.axis_index("c")
    sid = lax.axis_index("s")
    wid = sid * 2 + cid

    def block(t, carry):
        c = wid * BLK_PER_W + t

        @pl.when(c < NFULL)
        def _():
            pltpu.sync_copy(embt_hbm.at[:, pl.ds(c * 128, 128)], v_in)
            _transpose_block(v_in, v_out)
            pltpu.sync_copy(v_out, out_hbm.at[pl.ds(c * 32, 32)])
        return carry

    lax.fori_loop(0, BLK_PER_W, block, 0)

    @pl.when(wid == NUM_WORKERS - 1)
    def _():
        pltpu.sync_copy(tail_hbm, v_out)
        pltpu.sync_copy(v_out, out_hbm.at[pl.ds(NFULL * 32, 32)])


def _gather_body(x_hbm, offrep_hbm, table_hbm, out_hbm,
                 off_v, abs_v, idx_v, rows_a, rows_b, stage_v, sem_a, sem_b):
    cid = lax.axis_index("c")
    sid = lax.axis_index("s")
    wid = sid * 2 + cid
    base = wid * PER_W

    pltpu.sync_copy(offrep_hbm, off_v)
    bufs = (rows_a, rows_b)
    sems = (sem_a, sem_b)

    def fire(j, c0):
        return pltpu.async_copy(
            table_hbm.at[idx_v.at[pl.ds(j * GSZ, GSZ)]],
            bufs[j % 2],
            sems[j % 2],
        )

    def group(g2, carry):
        g_base = base + g2 * GROUP

        def chunk(cc, carry2):
            c_base = g_base + cc * CHUNK
            pltpu.sync_copy(x_hbm.at[pl.ds(c_base, CHUNK)], abs_v)
            o0 = cc * CHUNK
            for c in range(0, CHUNK, LANES):
                a = abs_v[pl.ds(c, LANES)] + off_v[pl.ds(o0 + c, LANES)]
                abs_v[pl.ds(c, LANES)] = a
                idx_v[pl.ds(c, LANES)] = lax.shift_right_logical(a, 2)

            def extract(j, rows_v):
                for g in range(GSZ // LANES):
                    jl = j * GSZ + g * LANES
                    jj = o0 + jl + lax.iota(jnp.int32, LANES)
                    col0 = lax.rem(abs_v[pl.ds(jl, LANES)], 4) * DIM
                    f_vec = lax.rem(jj, F)
                    b_vec = lax.div(jj, F)
                    lrow = g * LANES + lax.iota(jnp.int32, LANES)
                    for d in range(DIM):
                        v = plsc.load_gather(rows_v, [lrow, col0 + d])
                        plsc.store_scatter(
                            stage_v,
                            [f_vec, jnp.full((LANES,), d, jnp.int32), b_vec],
                            v)

            cps = [None, None]
            cps[0] = fire(0, o0)
            for j in range(1, NG):
                cps[j % 2] = fire(j, o0)
                cps[(j - 1) % 2].wait()
                extract(j - 1, bufs[(j - 1) % 2])
            cps[(NG - 1) % 2].wait()
            extract(NG - 1, bufs[(NG - 1) % 2])
            return carry2

        lax.fori_loop(0, NCHUNK, chunk, 0)
        b0 = wid * (PER_W // F) + g2 * 128
        pltpu.sync_copy(stage_v, out_hbm.at[:, :, pl.ds(b0, 128)])
        return carry

    lax.fori_loop(0, NGROUPS, group, 0)


@jax.jit
def _emb_call(x_flat, off_rep, emb_t, tail_packed):
    mesh = plsc.VectorSubcoreMesh(core_axis_name="c", subcore_axis_name="s")
    detile = pl.kernel(
        _detile_body,
        out_type=jax.ShapeDtypeStruct((Q, 128), jnp.float32),
        mesh=mesh,
        scratch_types=[
            pltpu.VMEM((32, 128), jnp.float32),
            pltpu.VMEM((32, 128), jnp.float32),
        ],
        compiler_params=_CPARAMS,
    )
    table = detile(emb_t, tail_packed)
    gather = pl.kernel(
        _gather_body,
        out_type=jax.ShapeDtypeStruct((F, DIM, B), jnp.float32),
        mesh=mesh,
        scratch_types=[
            pltpu.VMEM((GROUP,), jnp.int32),
            pltpu.VMEM((CHUNK,), jnp.int32),
            pltpu.VMEM((CHUNK,), jnp.int32),
            pltpu.VMEM((GSZ, 128), jnp.float32),
            pltpu.VMEM((GSZ, 128), jnp.float32),
            pltpu.VMEM((F, DIM, 128), jnp.float32),
            pltpu.SemaphoreType.DMA,
            pltpu.SemaphoreType.DMA,
        ],
        compiler_params=_CPARAMS,
    )
    return gather(x_flat, off_rep, table)


def kernel(x_cat, emb_weight, offsets):
    x_flat = x_cat.astype(jnp.int32).reshape(N)
    off_rep = jnp.tile(offsets.astype(jnp.int32), GROUP // F)
    tcol = jnp.zeros((128, DIM), jnp.float32)
    tcol = tcol.at[:TAIL].set(emb_weight[NFULL * 128:, :])
    tail_packed = tcol.reshape(32, 128)
    out = _emb_call(x_flat, off_rep, emb_weight.T, tail_packed)
    return jnp.transpose(out, (2, 0, 1))


# speedup vs baseline: 1.8425x; 1.1496x over previous
import jax
import jax.numpy as jnp
from jax import lax
from jax.experimental import pallas as pl
from jax.experimental.pallas import tpu as pltpu
from jax.experimental.pallas import tpu_sc as plsc

B = 16384
F = 26
DIM = 32
N = B * F
R = 2600001
RPAD = 2600064
NBLK = RPAD // 128
Q = RPAD // 4
NUM_WORKERS = 32
LANES = 16

NFULL = NBLK - 1
TAIL = R - NFULL * 128
BLK_PER_W = -(-NFULL // NUM_WORKERS)

PER_W = N // NUM_WORKERS
GROUP = 128 * F
NGROUPS = PER_W // GROUP
CHUNK = 832
NCHUNK = GROUP // CHUNK
GSZ = 64
NG = CHUNK // GSZ

_CPARAMS = pltpu.CompilerParams(
    use_tc_tiling_on_sc=True, needs_layout_passes=False)


SUP = 1
NSUPER = NFULL // SUP
SUP_PER_W = NSUPER // NUM_WORKERS
NREM = NSUPER - SUP_PER_W * NUM_WORKERS


def _transpose_super(v_in, v_out):
    for bb in range(SUP):
        for r in range(32):
            for k in range(8):
                rows = (k % 2) * LANES + lax.iota(jnp.int32, LANES)
                src = plsc.load_gather(
                    v_in,
                    [rows,
                     jnp.full((LANES,), bb * 128 + 4 * r + k // 2, jnp.int32)],
                )
                v_out[bb * 32 + r, pl.ds(k * LANES, LANES)] = src


def _detile_body(embt_hbm, tail_hbm, out_hbm,
                 in_a, in_b, out_a, out_b, sin_a, sin_b, sout_a, sout_b):
    cid = lax.axis_index("c")
    sid = lax.axis_index("s")
    wid = sid * 2 + cid
    vins = (in_a, in_b)
    vouts = (out_a, out_b)
    sins = (sin_a, sin_b)
    souts = (sout_a, sout_b)

    def fire_in(t, p):
        col = (t * NUM_WORKERS + wid) * (SUP * 128)
        return pltpu.async_copy(
            embt_hbm.at[:, pl.ds(col, SUP * 128)], vins[p], sins[p])

    def fire_out(t, p):
        row = (t * NUM_WORKERS + wid) * (SUP * 32)
        return pltpu.async_copy(
            vouts[p], out_hbm.at[pl.ds(row, SUP * 32)], souts[p])

    fire_in(0, 0)

    def pair(t2, carry):
        t = t2 * 2
        for p in range(2):
            @pl.when(t + p + 1 < SUP_PER_W)
            def _():
                fire_in(t + p + 1, 1 - p)
            pltpu.make_async_copy(
                embt_hbm.at[:, pl.ds(0, SUP * 128)], vins[p], sins[p]).wait()

            @pl.when(t + p >= 2)
            def _():
                pltpu.make_async_copy(
                    vouts[p], out_hbm.at[pl.ds(0, SUP * 32)], souts[p]).wait()
            _transpose_super(vins[p], vouts[p])
            fire_out(t + p, p)
        return carry

    lax.fori_loop(0, SUP_PER_W // 2, pair, 0)
    for p in range(2):
        pltpu.make_async_copy(
            vouts[p], out_hbm.at[pl.ds(0, SUP * 32)], souts[p]).wait()

    @pl.when(wid < NREM)
    def _():
        sidx = SUP_PER_W * NUM_WORKERS + wid
        pltpu.sync_copy(embt_hbm.at[:, pl.ds(sidx * (SUP * 128), SUP * 128)], in_a)
        _transpose_super(in_a, out_a)
        pltpu.sync_copy(out_a, out_hbm.at[pl.ds(sidx * (SUP * 32), SUP * 32)])

    @pl.when(wid == NUM_WORKERS - 1)
    def _():
        pltpu.sync_copy(tail_hbm, out_b.at[pl.ds(0, 32)])
        pltpu.sync_copy(out_b.at[pl.ds(0, 32)], out_hbm.at[pl.ds(NFULL * 32, 32)])


def _gather_body(x_hbm, offrep_hbm, table_hbm, out_hbm,
                 off_v, abs_v, idx_v, rows_a, rows_b, stage_v, sem_a, sem_b):
    cid = lax.axis_index("c")
    sid = lax.axis_index("s")
    wid = sid * 2 + cid
    base = wid * PER_W

    pltpu.sync_copy(offrep_hbm, off_v)
    bufs = (rows_a, rows_b)
    sems = (sem_a, sem_b)

    def fire(j, c0):
        return pltpu.async_copy(
            table_hbm.at[idx_v.at[pl.ds(j * GSZ, GSZ)]],
            bufs[j % 2],
            sems[j % 2],
        )

    def group(g2, carry):
        g_base = base + g2 * GROUP

        def chunk(cc, carry2):
            c_base = g_base + cc * CHUNK
            pltpu.sync_copy(x_hbm.at[pl.ds(c_base, CHUNK)], abs_v)
            o0 = cc * CHUNK
            for c in range(0, CHUNK, LANES):
                a = abs_v[pl.ds(c, LANES)] + off_v[pl.ds(o0 + c, LANES)]
                abs_v[pl.ds(c, LANES)] = a
                idx_v[pl.ds(c, LANES)] = lax.shift_right_logical(a, 2)

            def extract(j, rows_v):
                for g in range(GSZ // LANES):
                    jl = j * GSZ + g * LANES
                    jj = o0 + jl + lax.iota(jnp.int32, LANES)
                    col0 = lax.rem(abs_v[pl.ds(jl, LANES)], 4) * DIM
                    f_vec = lax.rem(jj, F)
                    b_vec = lax.div(jj, F)
                    lrow = g * LANES + lax.iota(jnp.int32, LANES)
                    for d in range(DIM):
                        v = plsc.load_gather(rows_v, [lrow, col0 + d])
                        plsc.store_scatter(
                            stage_v,
                            [f_vec, jnp.full((LANES,), d, jnp.int32), b_vec],
                            v)

            cps = [None, None]
            cps[0] = fire(0, o0)
            for j in range(1, NG):
                cps[j % 2] = fire(j, o0)
                cps[(j - 1) % 2].wait()
                extract(j - 1, bufs[(j - 1) % 2])
            cps[(NG - 1) % 2].wait()
            extract(NG - 1, bufs[(NG - 1) % 2])
            return carry2

        lax.fori_loop(0, NCHUNK, chunk, 0)
        b0 = wid * (PER_W // F) + g2 * 128
        pltpu.sync_copy(stage_v, out_hbm.at[:, :, pl.ds(b0, 128)])
        return carry

    lax.fori_loop(0, NGROUPS, group, 0)


@jax.jit
def _emb_call(x_flat, off_rep, emb_t, tail_packed):
    mesh = plsc.VectorSubcoreMesh(core_axis_name="c", subcore_axis_name="s")
    detile = pl.kernel(
        _detile_body,
        out_type=jax.ShapeDtypeStruct((Q, 128), jnp.float32),
        mesh=mesh,
        scratch_types=[
            pltpu.VMEM((32, SUP * 128), jnp.float32),
            pltpu.VMEM((32, SUP * 128), jnp.float32),
            pltpu.VMEM((SUP * 32, 128), jnp.float32),
            pltpu.VMEM((SUP * 32, 128), jnp.float32),
            pltpu.SemaphoreType.DMA,
            pltpu.SemaphoreType.DMA,
            pltpu.SemaphoreType.DMA,
            pltpu.SemaphoreType.DMA,
        ],
        compiler_params=_CPARAMS,
    )
    table = detile(emb_t, tail_packed)
    gather = pl.kernel(
        _gather_body,
        out_type=jax.ShapeDtypeStruct((F, DIM, B), jnp.float32),
        mesh=mesh,
        scratch_types=[
            pltpu.VMEM((GROUP,), jnp.int32),
            pltpu.VMEM((CHUNK,), jnp.int32),
            pltpu.VMEM((CHUNK,), jnp.int32),
            pltpu.VMEM((GSZ, 128), jnp.float32),
            pltpu.VMEM((GSZ, 128), jnp.float32),
            pltpu.VMEM((F, DIM, 128), jnp.float32),
            pltpu.SemaphoreType.DMA,
            pltpu.SemaphoreType.DMA,
        ],
        compiler_params=_CPARAMS,
    )
    return gather(x_flat, off_rep, table)


def kernel(x_cat, emb_weight, offsets):
    x_flat = x_cat.astype(jnp.int32).reshape(N)
    off_rep = jnp.tile(offsets.astype(jnp.int32), GROUP // F)
    tcol = jnp.zeros((128, DIM), jnp.float32)
    tcol = tcol.at[:TAIL].set(emb_weight[NFULL * 128:, :])
    tail_packed = tcol.reshape(32, 128)
    out = _emb_call(x_flat, off_rep, emb_weight.T, tail_packed)
    return jnp.transpose(out, (2, 0, 1))


# speedup vs baseline: 3.2335x; 1.7550x over previous
import jax
import jax.numpy as jnp
from jax import lax
from jax.experimental import pallas as pl
from jax.experimental.pallas import tpu as pltpu
from jax.experimental.pallas import tpu_sc as plsc

B = 16384
F = 26
DIM = 32
N = B * F
NUM_WORKERS = 32
PER_W = N // NUM_WORKERS
GSZ = 128
K = 13
STEP = K * GSZ
NSTEPS = PER_W // STEP
LANES = 16


def _emb_body(x_hbm, offrep_hbm, table_hbm, out_hbm,
              off_v, raw_v, idx_v, rows_v, sem):
    cid = lax.axis_index("c")
    sid = lax.axis_index("s")
    wid = sid * 2 + cid
    base = wid * PER_W

    pltpu.sync_copy(offrep_hbm, off_v)

    def step(s, carry):
        s_base = base + s * STEP
        pltpu.sync_copy(x_hbm.at[pl.ds(s_base, STEP)], raw_v)
        for j in range(K):
            for c in range(GSZ // LANES):
                p0 = j * GSZ + c * LANES
                idx_v[j, pl.ds(c * LANES, LANES)] = (
                    raw_v[pl.ds(p0, LANES)] + off_v[pl.ds(p0, LANES)]
                )
        cps = [
            pltpu.async_copy(
                table_hbm.at[idx_v.at[j]],
                rows_v.at[pl.ds(j * GSZ, GSZ)],
                sem,
            )
            for j in range(K)
        ]
        for cp in cps:
            cp.wait()
        pltpu.sync_copy(rows_v, out_hbm.at[pl.ds(s_base, STEP)])
        return carry

    lax.fori_loop(0, NSTEPS, step, 0)


@jax.jit
def _emb_call(x_flat, off_rep, table):
    mesh = plsc.VectorSubcoreMesh(core_axis_name="c", subcore_axis_name="s")
    f = pl.kernel(
        _emb_body,
        out_type=jax.ShapeDtypeStruct((N, DIM), jnp.float32),
        mesh=mesh,
        scratch_types=[
            pltpu.VMEM((STEP,), jnp.int32),
            pltpu.VMEM((STEP,), jnp.int32),
            pltpu.VMEM((K, GSZ), jnp.int32),
            pltpu.VMEM((STEP, DIM), jnp.float32),
            pltpu.SemaphoreType.DMA,
        ],
        compiler_params=pltpu.CompilerParams(use_tc_tiling_on_sc=False),
    )
    return f(x_flat, off_rep, table)


def kernel(x_cat, emb_weight, offsets):
    x_flat = x_cat.astype(jnp.int32).reshape(N)
    off_rep = jnp.tile(offsets.astype(jnp.int32), STEP // F)
    out = _emb_call(x_flat, off_rep, emb_weight)
    return out.reshape(B, F, DIM)
